# baseline (speedup 1.0000x reference)
import numpy as np
import jax
import jax.numpy as jnp
from jax import lax
from jax.experimental import pallas as pl
from jax.experimental.pallas import tpu as pltpu
from jax.experimental.pallas import tpu_sc as plsc

VOCAB = 1000000
D = 64
SEQ = 200
BATCH = 4096
NC = 2
NS = 16
NW = NC * NS
BLK = BATCH // NW
L16 = 16
SCALE = 8.0


def _pos_encoding(length, depth):
    half = depth // 2
    positions = np.arange(length)[:, np.newaxis]
    depths = np.arange(half)[np.newaxis, :] / half
    angle_rates = 1 / 10000 ** depths
    angle_rads = positions * angle_rates
    pe = np.concatenate([np.sin(angle_rads), np.cos(angle_rads)], axis=-1)
    return pe.astype(np.float32)


def _sc_kernel(t2, xt, posx, out, xv, gidx, pv, gbuf, posv, obuf,
               gsem, osem, psem, xsem):
    cid = lax.axis_index("c")
    sid = lax.axis_index("s")
    wid = sid * NC + cid
    b0 = wid * BLK

    def fetch_x(l, b):
        return pltpu.make_async_copy(
            xt.at[l, pl.ds(b0, BLK)], xv.at[b], xsem.at[b])

    def fetch_pos(l, b):
        return pltpu.make_async_copy(posx.at[l], posv.at[b], psem.at[b])

    def gather(b):
        return pltpu.make_async_copy(t2.at[gidx.at[b]], gbuf.at[b],
                                     gsem.at[b])

    def prep_idx(b):
        for t in range(BLK // L16):
            s = pl.ds(t * L16, L16)
            v = xv[b, s]
            gidx[b, s] = lax.shift_right_logical(v, 1)
            pv[b, s] = lax.shift_left((v & 1), 6)

    def put(l, b):
        return pltpu.make_async_copy(
            obuf.at[b], out.at[l, :, pl.ds(b0, BLK)], osem.at[b])

    fetch_x(0, 0).start()
    fetch_pos(0, 0).start()
    fetch_x(1, 1).start()
    fetch_pos(1, 1).start()
    fetch_x(0, 0).wait()
    prep_idx(0)
    gather(0).start()

    jts = [lax.iota(jnp.int32, L16) + (t * L16) for t in range(8)]

    def do_unit(l, b):
        @pl.when(l + 2 < SEQ)
        def _():
            fetch_x(l + 2, b).start()

        @pl.when(l + 1 < SEQ)
        def _():
            fetch_x(l + 1, 1 - b).wait()
            prep_idx(1 - b)
            gather(1 - b).start()

        gather(b).wait()
        fetch_pos(l, b).wait()

        @pl.when(l >= 2)
        def _():
            put(l - 2, b).wait()

        pvs0 = tuple(pv[b, pl.ds(t * L16, L16)] for t in range(8))

        @plsc.parallel_loop(0, D, carry=pvs0, unroll=4)
        def col(d, pvs):
            p = posv[b, d]
            for t in range(8):
                cols = pvs[t] + d
                g = plsc.load_gather(gbuf.at[b], [jts[t], cols])
                obuf[b, d, pl.ds(t * L16, L16)] = g * SCALE + p
            return pvs

        put(l, b).start()

        @pl.when(l + 2 < SEQ)
        def _():
            fetch_pos(l + 2, b).start()

    def pair(k, carry):
        l0 = k * 2
        do_unit(l0, 0)
        do_unit(l0 + 1, 1)
        return carry

    lax.fori_loop(0, SEQ // 2, pair, 0)

    put(SEQ - 2, 0).wait()
    put(SEQ - 1, 1).wait()


def kernel(x, embedding_table):
    x = x.astype(jnp.int32)
    t2 = embedding_table.reshape(VOCAB // 2, 2 * D)
    xt = x.T
    posn = _pos_encoding(2048, D)[:SEQ]
    posx = jnp.asarray(np.repeat(posn[:, :, None], L16, axis=2))

    mesh = plsc.VectorSubcoreMesh(
        core_axis_name="c", subcore_axis_name="s",
        num_cores=NC, num_subcores=NS)

    out_t = pl.kernel(
        _sc_kernel,
        out_type=jax.ShapeDtypeStruct((SEQ, D, BATCH), jnp.float32),
        mesh=mesh,
        compiler_params=pltpu.CompilerParams(
            use_tc_tiling_on_sc=True, needs_layout_passes=False),
        scratch_types=[
            pltpu.VMEM((2, BLK), jnp.int32),
            pltpu.VMEM((2, BLK), jnp.int32),
            pltpu.VMEM((2, BLK), jnp.int32),
            pltpu.VMEM((2, BLK, 2 * D), jnp.float32),
            pltpu.VMEM((2, D, L16), jnp.float32),
            pltpu.VMEM((2, D, BLK), jnp.float32),
            pltpu.SemaphoreType.DMA((2,)),
            pltpu.SemaphoreType.DMA((2,)),
            pltpu.SemaphoreType.DMA((2,)),
            pltpu.SemaphoreType.DMA((2,)),
        ],
    )(t2, xt, posx)

    return jnp.transpose(out_t, (2, 0, 1))


# speedup vs baseline: 1.0418x; 1.0418x over previous
import numpy as np
import jax
import jax.numpy as jnp
from jax import lax
from jax.experimental import pallas as pl
from jax.experimental.pallas import tpu as pltpu
from jax.experimental.pallas import tpu_sc as plsc

VOCAB = 1000000
D = 64
SEQ = 200
BATCH = 4096
NC = 2
NS = 16
NW = NC * NS
ROWS_PER_W = BATCH // NW
SPLIT_A = 120
SPLIT_B = 80
NBUF_IN = 4
NBUF_OUT = 2
SCALE = 8.0


def _pos_encoding(length, depth):
    half = depth // 2
    positions = np.arange(length)[:, np.newaxis]
    depths = np.arange(half)[np.newaxis, :] / half
    angle_rates = 1 / 10000 ** depths
    angle_rads = positions * angle_rates
    pe = np.concatenate([np.sin(angle_rads), np.cos(angle_rads)], axis=-1)
    return pe.astype(np.float32)


def _sc_kernel(table, x, pos_h, out, idxa, idxb, bin_, bout, pos_v,
               gsem, ssem):
    cid = lax.axis_index("c")
    sid = lax.axis_index("s")
    wid = sid * NC + cid
    r0 = wid * ROWS_PER_W

    pltpu.sync_copy(pos_h, pos_v)
    pltpu.sync_copy(x.at[pl.ds(r0, ROWS_PER_W), pl.ds(0, SPLIT_A)], idxa)
    pltpu.sync_copy(x.at[pl.ds(r0, ROWS_PER_W), pl.ds(SPLIT_A, SPLIT_B)],
                    idxb)

    def gather_a(g, b):
        return pltpu.make_async_copy(
            table.at[idxa.at[g]], bin_.at[b, pl.ds(0, SPLIT_A)], gsem.at[b])

    def gather_b(g, b):
        return pltpu.make_async_copy(
            table.at[idxb.at[g]], bin_.at[b, pl.ds(SPLIT_A, SPLIT_B)],
            gsem.at[b])

    def scatter(g, ob):
        return pltpu.make_async_copy(
            bout.at[ob], out.at[pl.ds((r0 + g) * SEQ, SEQ)], ssem.at[ob])

    for b in range(NBUF_IN):
        gather_a(b, b).start()
        gather_b(b, b).start()

    def outer(k, carry):
        g0 = k * NBUF_IN
        for b in range(NBUF_IN):
            g = g0 + b
            ob = b % NBUF_OUT
            gather_a(g, b).wait()
            gather_b(g, b).wait()

            @pl.when(g >= NBUF_OUT)
            def _():
                scatter(g - NBUF_OUT, ob).wait()

            def row(i, c):
                for d in range(D // 16):
                    s = pl.ds(d * 16, 16)
                    bout[ob, i, s] = bin_[b, i, s] * SCALE + pos_v[i, s]
                return c

            lax.fori_loop(0, SEQ, row, 0)
            scatter(g, ob).start()

            @pl.when(g + NBUF_IN < ROWS_PER_W)
            def _():
                gather_a(g + NBUF_IN, b).start()
                gather_b(g + NBUF_IN, b).start()
        return carry

    lax.fori_loop(0, ROWS_PER_W // NBUF_IN, outer, 0)

    for t in range(NBUF_OUT):
        g = ROWS_PER_W - NBUF_OUT + t
        scatter(g, g % NBUF_OUT).wait()


def kernel(x, embedding_table):
    x = x.astype(jnp.int32)
    pos = jnp.asarray(_pos_encoding(2048, D)[:SEQ])

    mesh = plsc.VectorSubcoreMesh(
        core_axis_name="c", subcore_axis_name="s",
        num_cores=NC, num_subcores=NS)

    out = pl.kernel(
        _sc_kernel,
        out_type=jax.ShapeDtypeStruct((BATCH * SEQ, D), jnp.float32),
        mesh=mesh,
        compiler_params=pltpu.CompilerParams(use_tc_tiling_on_sc=False),
        scratch_types=[
            pltpu.VMEM((ROWS_PER_W, SPLIT_A), jnp.int32),
            pltpu.VMEM((ROWS_PER_W, SPLIT_B), jnp.int32),
            pltpu.VMEM((NBUF_IN, SEQ, D), jnp.float32),
            pltpu.VMEM((NBUF_OUT, SEQ, D), jnp.float32),
            pltpu.VMEM((SEQ, D), jnp.float32),
            pltpu.SemaphoreType.DMA((NBUF_IN,)),
            pltpu.SemaphoreType.DMA((NBUF_OUT,)),
        ],
    )(embedding_table, x, pos)

    return out.reshape(BATCH, SEQ, D)


# speedup vs baseline: 1.0430x; 1.0012x over previous
import numpy as np
import jax
import jax.numpy as jnp
from jax import lax
from jax.experimental import pallas as pl
from jax.experimental.pallas import tpu as pltpu
from jax.experimental.pallas import tpu_sc as plsc

VOCAB = 1000000
D = 64
SEQ = 200
BATCH = 4096
NC = 2
NS = 16
NW = NC * NS
ROWS_PER_W = BATCH // NW
SPLIT_A = 120
SPLIT_B = 80
NBUF_IN = 4
NBUF_OUT = 2
SCALE = 8.0


def _pos_encoding(length, depth):
    half = depth // 2
    positions = np.arange(length)[:, np.newaxis]
    depths = np.arange(half)[np.newaxis, :] / half
    angle_rates = 1 / 10000 ** depths
    angle_rads = positions * angle_rates
    pe = np.concatenate([np.sin(angle_rads), np.cos(angle_rads)], axis=-1)
    return pe.astype(np.float32)


def _sc_kernel(table, x, pos_h, out, idxa, idxb, bin_, bout, pos_v,
               gsema, gsemb, ssem):
    cid = lax.axis_index("c")
    sid = lax.axis_index("s")
    wid = sid * NC + cid
    r0 = wid * ROWS_PER_W

    pltpu.sync_copy(pos_h, pos_v)
    pltpu.sync_copy(x.at[pl.ds(r0, ROWS_PER_W), pl.ds(0, SPLIT_A)], idxa)
    pltpu.sync_copy(x.at[pl.ds(r0, ROWS_PER_W), pl.ds(SPLIT_A, SPLIT_B)],
                    idxb)

    def gather_a(g, b):
        return pltpu.make_async_copy(
            table.at[idxa.at[g]], bin_.at[b, pl.ds(0, SPLIT_A)], gsema.at[b])

    def gather_b(g, b):
        return pltpu.make_async_copy(
            table.at[idxb.at[g]], bin_.at[b, pl.ds(SPLIT_A, SPLIT_B)],
            gsemb.at[b])

    def scatter(g, ob):
        return pltpu.make_async_copy(
            bout.at[ob], out.at[pl.ds((r0 + g) * SEQ, SEQ)], ssem.at[ob])

    for b in range(NBUF_IN):
        gather_a(b, b).start()
        gather_b(b, b).start()

    def outer(k, carry):
        g0 = k * NBUF_IN
        for b in range(NBUF_IN):
            g = g0 + b
            ob = b % NBUF_OUT
            gather_a(g, b).wait()
            gather_b(g, b).wait()

            @pl.when(g >= NBUF_OUT)
            def _():
                scatter(g - NBUF_OUT, ob).wait()

            def row(i, c):
                for d in range(D // 16):
                    s = pl.ds(d * 16, 16)
                    bout[ob, i, s] = bin_[b, i, s] * SCALE + pos_v[i, s]
                return c

            lax.fori_loop(0, SEQ, row, 0)
            scatter(g, ob).start()

            @pl.when(g + NBUF_IN < ROWS_PER_W)
            def _():
                gather_a(g + NBUF_IN, b).start()
                gather_b(g + NBUF_IN, b).start()
        return carry

    lax.fori_loop(0, ROWS_PER_W // NBUF_IN, outer, 0)

    for t in range(NBUF_OUT):
        g = ROWS_PER_W - NBUF_OUT + t
        scatter(g, g % NBUF_OUT).wait()


def kernel(x, embedding_table):
    x = x.astype(jnp.int32)
    pos = jnp.asarray(_pos_encoding(2048, D)[:SEQ])

    mesh = plsc.VectorSubcoreMesh(
        core_axis_name="c", subcore_axis_name="s",
        num_cores=NC, num_subcores=NS)

    out = pl.kernel(
        _sc_kernel,
        out_type=jax.ShapeDtypeStruct((BATCH * SEQ, D), jnp.float32),
        mesh=mesh,
        compiler_params=pltpu.CompilerParams(use_tc_tiling_on_sc=False),
        scratch_types=[
            pltpu.VMEM((ROWS_PER_W, SPLIT_A), jnp.int32),
            pltpu.VMEM((ROWS_PER_W, SPLIT_B), jnp.int32),
            pltpu.VMEM((NBUF_IN, SEQ, D), jnp.float32),
            pltpu.VMEM((NBUF_OUT, SEQ, D), jnp.float32),
            pltpu.VMEM((SEQ, D), jnp.float32),
            pltpu.SemaphoreType.DMA((NBUF_IN,)),
            pltpu.SemaphoreType.DMA((NBUF_IN,)),
            pltpu.SemaphoreType.DMA((NBUF_OUT,)),
        ],
    )(embedding_table, x, pos)

    return out.reshape(BATCH, SEQ, D)
